# Initial kernel scaffold; baseline (speedup 1.0000x reference)
import jax
import jax.numpy as jnp
from jax.experimental import pallas as pl


def kernel(parameters, dW1, db1, dg, dbeta, dW2, db2, fW1, fb1, fg, fbeta, fW2, fb2):
    raise NotImplementedError("write your pallas kernel here")


# speedup vs baseline: 24.8780x; 24.8780x over previous
import functools

import numpy as np
import jax
import jax.numpy as jnp
from jax import lax
from jax.experimental import pallas as pl
from jax.experimental.pallas import tpu as pltpu
from jax.experimental.pallas import tpu_sc as plsc

N = 8388608
NC, NS, L = 2, 16, 16
NW = NC * NS
CHUNK = N // NW
BLK = 8192
NBLK = CHUNK // BLK
VPB = BLK // L

L1_BINS = 4096
L2_BINS = 1024
MASK31 = np.int32(0x7FFFFFFF)
I32MAX = np.int32(2147483647)

_QF = np.float32([0.1, 0.5, 0.9])
_QIDX = (_QF * (np.float32(N) - np.float32(1.0))).astype(np.float32)
RANK_LO = [int(x) for x in np.floor(_QIDX)]
RANK_HI = [int(x) for x in np.ceil(_QIDX)]
HIW = (_QIDX - np.floor(_QIDX)).astype(np.float32)
LOW = (np.float32(1.0) - HIW).astype(np.float32)

_mesh = plsc.VectorSubcoreMesh(core_axis_name="c", subcore_axis_name="s")


def _wid():
    return lax.axis_index("s") * NC + lax.axis_index("c")


def _keys(x):
    b = plsc.bitcast(x, jnp.int32)
    return jnp.where(b < 0, b ^ MASK31, b)


@functools.partial(
    pl.kernel,
    out_type=(
        jax.ShapeDtypeStruct((NW, L1_BINS), jnp.int32),
        jax.ShapeDtypeStruct((NW, 64), jnp.float32),
    ),
    mesh=_mesh,
    compiler_params=pltpu.CompilerParams(needs_layout_passes=False),
    scratch_types=[
        pltpu.VMEM((BLK,), jnp.float32),
        pltpu.VMEM((BLK,), jnp.float32),
        pltpu.VMEM((L1_BINS * L,), jnp.int32),
        pltpu.VMEM((L1_BINS,), jnp.int32),
        pltpu.VMEM((64,), jnp.float32),
        pltpu.SemaphoreType.DMA,
        pltpu.SemaphoreType.DMA,
    ],
)
def _pass1(params, hist_out, mom_out, buf0, buf1, hist, merged, momv, sem0, sem1):
    wid = _wid()
    base = wid * CHUNK
    lane = lax.iota(jnp.int32, L)
    lane_off = lane * L1_BINS
    ones = jnp.ones((L,), jnp.int32)
    zeros16 = jnp.zeros((L,), jnp.int32)

    def zbody(i, _):
        for u in range(8):
            hist[pl.ds((i * 8 + u) * L, L)] = zeros16
        return 0
    lax.fori_loop(0, L1_BINS * L // (L * 8), zbody, 0)

    bufs = (buf0, buf1)
    sems = (sem0, sem1)
    cps = [None, None]
    cps[0] = pltpu.async_copy(params.at[pl.ds(base, BLK)], buf0, sem0)

    def make_body(buf):
        def body(i, carry):
            mnv, mxv, smv, sqv = carry
            x = buf[pl.ds(i * L, L)]
            key = _keys(x)
            idx = ((key >> 20) + jnp.int32(2048)) + lane_off
            plsc.addupdate_scatter(hist, [idx], ones)
            return (jnp.minimum(mnv, x), jnp.maximum(mxv, x),
                    smv + x, sqv + x * x)
        return body

    carry = (jnp.full((L,), jnp.inf, jnp.float32),
             jnp.full((L,), -jnp.inf, jnp.float32),
             jnp.zeros((L,), jnp.float32), jnp.zeros((L,), jnp.float32))
    for bkt in range(NBLK):
        cur = bkt % 2
        if bkt + 1 < NBLK:
            cps[1 - cur] = pltpu.async_copy(
                params.at[pl.ds(base + (bkt + 1) * BLK, BLK)],
                bufs[1 - cur], sems[1 - cur])
        cps[cur].wait()
        carry = lax.fori_loop(0, VPB, make_body(bufs[cur]), carry)

    mnv, mxv, smv, sqv = carry
    momv[pl.ds(0, L)] = mnv
    momv[pl.ds(L, L)] = mxv
    momv[pl.ds(2 * L, L)] = smv
    momv[pl.ds(3 * L, L)] = sqv
    pltpu.sync_copy(momv, mom_out.at[wid])

    def mbody(i, _):
        acc = hist[pl.ds(i * L, L)]
        for l in range(1, L):
            acc = acc + hist[pl.ds(l * L1_BINS + i * L, L)]
        merged[pl.ds(i * L, L)] = acc
        return 0
    lax.fori_loop(0, L1_BINS // L, mbody, 0)
    pltpu.sync_copy(merged, hist_out.at[wid])


@functools.partial(
    pl.kernel,
    out_type=(
        jax.ShapeDtypeStruct((NW, 3 * L2_BINS), jnp.int32),
        jax.ShapeDtypeStruct((NW, 16), jnp.int32),
    ),
    mesh=_mesh,
    compiler_params=pltpu.CompilerParams(needs_layout_passes=False),
    scratch_types=[
        pltpu.VMEM((BLK,), jnp.float32),
        pltpu.VMEM((BLK,), jnp.float32),
        pltpu.VMEM((3 * L2_BINS * L,), jnp.int32),
        pltpu.VMEM((16 * L,), jnp.int32),
        pltpu.VMEM((3 * L2_BINS,), jnp.int32),
        pltpu.VMEM((16,), jnp.int32),
        pltpu.VMEM((16,), jnp.int32),
        pltpu.VMEM((16,), jnp.float32),
        pltpu.SemaphoreType.DMA,
        pltpu.SemaphoreType.DMA,
    ],
)
def _pass2(params, scal_i, scal_f, hist_out, ehist_out,
           buf0, buf1, chist, ehist, merged, emerged, sbi, sbf, sem0, sem1):
    wid = _wid()
    base = wid * CHUNK
    lane = lax.iota(jnp.int32, L)
    ones = jnp.ones((L,), jnp.int32)
    zeros16 = jnp.zeros((L,), jnp.int32)

    pltpu.sync_copy(scal_i, sbi)
    pltpu.sync_copy(scal_f, sbf)
    sv_i = sbi[pl.ds(0, L)]
    sv_f = sbf[pl.ds(0, L)]
    p12_0 = sv_i[0]
    p12_1 = sv_i[1]
    p12_2 = sv_i[2]
    mn = sv_f[0]
    inv_w = sv_f[1]

    def zbody(i, _):
        for u in range(8):
            chist[pl.ds((i * 8 + u) * L, L)] = zeros16
        return 0
    lax.fori_loop(0, 3 * L2_BINS * L // (L * 8), zbody, 0)
    def zebody(i, _):
        ehist[pl.ds(i * L, L)] = zeros16
        return 0
    lax.fori_loop(0, 16, zebody, 0)

    bufs = (buf0, buf1)
    sems = (sem0, sem1)
    cps = [None, None]
    cps[0] = pltpu.async_copy(params.at[pl.ds(base, BLK)], buf0, sem0)

    lane_off2 = lane * L2_BINS
    lane_off_e = lane * 16

    def make_body(buf):
        def body(i, _):
            x = buf[pl.ds(i * L, L)]
            key = _keys(x)
            hi12 = key >> 20
            bin10 = (key >> 10) & jnp.int32(1023)
            idx0 = bin10 + lane_off2
            plsc.addupdate_scatter(chist, [idx0], ones,
                                   mask=hi12 == p12_0)
            plsc.addupdate_scatter(chist, [idx0 + jnp.int32(L2_BINS * L)],
                                   ones, mask=hi12 == p12_1)
            plsc.addupdate_scatter(chist, [idx0 + jnp.int32(2 * L2_BINS * L)],
                                   ones, mask=hi12 == p12_2)
            t = (x - mn) * inv_w
            ie = jnp.clip(t.astype(jnp.int32), 0, 15)
            plsc.addupdate_scatter(ehist, [ie + lane_off_e], ones)
            return 0
        return body

    for bkt in range(NBLK):
        cur = bkt % 2
        if bkt + 1 < NBLK:
            cps[1 - cur] = pltpu.async_copy(
                params.at[pl.ds(base + (bkt + 1) * BLK, BLK)],
                bufs[1 - cur], sems[1 - cur])
        cps[cur].wait()
        lax.fori_loop(0, VPB, make_body(bufs[cur]), 0)

    eacc = ehist[pl.ds(0, L)]
    for l in range(1, L):
        eacc = eacc + ehist[pl.ds(l * 16, L)]
    emerged[pl.ds(0, L)] = eacc
    pltpu.sync_copy(emerged, ehist_out.at[wid])

    def mbody(i, _):
        for c in range(3):
            cb = c * L2_BINS * L
            acc = chist[pl.ds(cb + i * L, L)]
            for l in range(1, L):
                acc = acc + chist[pl.ds(cb + l * L2_BINS + i * L, L)]
            merged[pl.ds(c * L2_BINS + i * L, L)] = acc
        return 0
    lax.fori_loop(0, L2_BINS // L, mbody, 0)
    pltpu.sync_copy(merged, hist_out.at[wid])


@functools.partial(
    pl.kernel,
    out_type=(
        jax.ShapeDtypeStruct((NW, 3 * L2_BINS), jnp.int32),
        jax.ShapeDtypeStruct((NW, 48), jnp.int32),
    ),
    mesh=_mesh,
    compiler_params=pltpu.CompilerParams(needs_layout_passes=False),
    scratch_types=[
        pltpu.VMEM((BLK,), jnp.float32),
        pltpu.VMEM((BLK,), jnp.float32),
        pltpu.VMEM((3 * L2_BINS * L,), jnp.int32),
        pltpu.VMEM((3 * L2_BINS,), jnp.int32),
        pltpu.VMEM((48,), jnp.int32),
        pltpu.VMEM((16,), jnp.int32),
        pltpu.SemaphoreType.DMA,
        pltpu.SemaphoreType.DMA,
    ],
)
def _pass3(params, scal_i, hist_out, mink_out,
           buf0, buf1, chist, merged, minkv, sbi, sem0, sem1):
    wid = _wid()
    base = wid * CHUNK
    lane = lax.iota(jnp.int32, L)
    ones = jnp.ones((L,), jnp.int32)
    zeros16 = jnp.zeros((L,), jnp.int32)

    pltpu.sync_copy(scal_i, sbi)
    sv_i = sbi[pl.ds(0, L)]
    p22_0 = sv_i[0]
    p22_1 = sv_i[1]
    p22_2 = sv_i[2]

    def zbody(i, _):
        for u in range(8):
            chist[pl.ds((i * 8 + u) * L, L)] = zeros16
        return 0
    lax.fori_loop(0, 3 * L2_BINS * L // (L * 8), zbody, 0)

    bufs = (buf0, buf1)
    sems = (sem0, sem1)
    cps = [None, None]
    cps[0] = pltpu.async_copy(params.at[pl.ds(base, BLK)], buf0, sem0)

    lane_off2 = lane * L2_BINS

    def make_body(buf):
        def body(i, carry):
            mk0, mk1, mk2 = carry
            x = buf[pl.ds(i * L, L)]
            key = _keys(x)
            hi22 = key >> 10
            idx0 = (key & jnp.int32(1023)) + lane_off2
            plsc.addupdate_scatter(chist, [idx0], ones,
                                   mask=hi22 == p22_0)
            plsc.addupdate_scatter(chist, [idx0 + jnp.int32(L2_BINS * L)],
                                   ones, mask=hi22 == p22_1)
            plsc.addupdate_scatter(chist, [idx0 + jnp.int32(2 * L2_BINS * L)],
                                   ones, mask=hi22 == p22_2)
            mk0 = jnp.minimum(mk0, jnp.where(hi22 > p22_0, key, I32MAX))
            mk1 = jnp.minimum(mk1, jnp.where(hi22 > p22_1, key, I32MAX))
            mk2 = jnp.minimum(mk2, jnp.where(hi22 > p22_2, key, I32MAX))
            return (mk0, mk1, mk2)
        return body

    carry = (jnp.full((L,), I32MAX, jnp.int32),) * 3
    for bkt in range(NBLK):
        cur = bkt % 2
        if bkt + 1 < NBLK:
            cps[1 - cur] = pltpu.async_copy(
                params.at[pl.ds(base + (bkt + 1) * BLK, BLK)],
                bufs[1 - cur], sems[1 - cur])
        cps[cur].wait()
        carry = lax.fori_loop(0, VPB, make_body(bufs[cur]), carry)

    minkv[pl.ds(0, L)] = carry[0]
    minkv[pl.ds(L, L)] = carry[1]
    minkv[pl.ds(2 * L, L)] = carry[2]
    pltpu.sync_copy(minkv, mink_out.at[wid])

    def mbody(i, _):
        for c in range(3):
            cb = c * L2_BINS * L
            acc = chist[pl.ds(cb + i * L, L)]
            for l in range(1, L):
                acc = acc + chist[pl.ds(cb + l * L2_BINS + i * L, L)]
            merged[pl.ds(c * L2_BINS + i * L, L)] = acc
        return 0
    lax.fori_loop(0, L2_BINS // L, mbody, 0)
    pltpu.sync_copy(merged, hist_out.at[wid])


def _heads_body(ecnt_ref, scal_ref,
                dW1_ref, db1_ref, dg_ref, dbeta_ref, dW2_ref, db2_ref,
                fW1_ref, fb1_ref, fg_ref, fbeta_ref, fW2_ref, fb2_ref,
                dens_ref, fish_ref, ent_ref):
    counts = ecnt_ref[0:1, :]
    probs = counts * jnp.float32(1.0 / N)
    logp = jnp.log(jnp.where(probs > 0, probs, jnp.float32(1.0)))
    ent = -jnp.sum(jnp.where(probs > 0, probs * logp, jnp.float32(0.0)))
    norm_ent = ent * jnp.float32(1.4426950408889634 / 4.0)
    ent_ref[...] = jnp.reshape(norm_ent, (1, 1))

    lanes = lax.broadcasted_iota(jnp.int32, (8, 128), 1)
    feat_f = jnp.broadcast_to(scal_ref[0:1, :], (8, 128))
    feat_d = jnp.where(lanes == 7, norm_ent, feat_f)

    def head(feat, W1, b1, g, beta, W2, b2):
        h = lax.dot_general(feat, W1[...], (((1,), (1,)), ((), ())),
                            preferred_element_type=jnp.float32)
        h = h + b1[0:1, :]
        mu = jnp.mean(h, axis=-1, keepdims=True)
        var = jnp.mean((h - mu) ** 2, axis=-1, keepdims=True)
        h = (h - mu) / jnp.sqrt(var + jnp.float32(1e-5)) * g[0:1, :] + beta[0:1, :]
        h = jnp.float32(0.5) * h * (jnp.float32(1.0) +
                                    lax.erf(h * jnp.float32(0.7071067811865476)))
        o = lax.dot_general(h, W2[...], (((1,), (1,)), ((), ())),
                            preferred_element_type=jnp.float32)
        return o[0:1, 0:1] + b2[0:1, 0:1]

    of = head(feat_f, fW1_ref, fb1_ref, fg_ref, fbeta_ref, fW2_ref, fb2_ref)
    fish_ref[...] = (jnp.maximum(of, 0.0) +
                     jnp.log1p(jnp.exp(-jnp.abs(of))))

    od = head(feat_d, dW1_ref, db1_ref, dg_ref, dbeta_ref, dW2_ref, db2_ref)
    dens_ref[...] = jnp.float32(1.0) / (jnp.float32(1.0) + jnp.exp(-od))


_heads = pl.pallas_call(
    _heads_body,
    out_shape=(
        jax.ShapeDtypeStruct((1, 1), jnp.float32),
        jax.ShapeDtypeStruct((1, 1), jnp.float32),
        jax.ShapeDtypeStruct((1, 1), jnp.float32),
    ),
)


def _first_ge(cum, target):
    return jnp.sum((cum < target).astype(jnp.int32), axis=-1)


def _unkey(k):
    return lax.bitcast_convert_type(jnp.where(k < 0, k ^ MASK31, k),
                                    jnp.float32)


def kernel(parameters, dW1, db1, dg, dbeta, dW2, db2,
           fW1, fb1, fg, fbeta, fW2, fb2):
    ranks = jnp.array(RANK_LO, jnp.int32)

    hist1, mom = _pass1(parameters)
    mom = mom.reshape(NW, 4, L)
    mn = jnp.min(mom[:, 0, :])
    mx = jnp.max(mom[:, 1, :])
    sm = jnp.sum(mom[:, 2, :])
    sq = jnp.sum(mom[:, 3, :])
    mean = sm / jnp.float32(N)
    var = (sq - sm * sm / jnp.float32(N)) / jnp.float32(N - 1)
    std = jnp.sqrt(var)

    h1 = jnp.sum(hist1, axis=0)
    cum1 = jnp.cumsum(h1)
    b1 = _first_ge(cum1[None, :], (ranks + 1)[:, None])
    cb1 = cum1[b1] - h1[b1]
    r1 = ranks - cb1
    p12 = b1 - 2048

    scal_i2 = jnp.zeros((16,), jnp.int32).at[0:3].set(p12)
    inv_w = jnp.float32(16.0) / (mx - mn)
    scal_f2 = jnp.zeros((16,), jnp.float32).at[0].set(mn).at[1].set(inv_w)

    hist2, ehist = _pass2(parameters, scal_i2, scal_f2)
    h2 = jnp.sum(hist2, axis=0).reshape(3, L2_BINS)
    cum2 = jnp.cumsum(h2, axis=1)
    b2 = _first_ge(cum2, (r1 + 1)[:, None])
    h2b = jnp.take_along_axis(h2, b2[:, None], axis=1)[:, 0]
    c2b = jnp.take_along_axis(cum2, b2[:, None], axis=1)[:, 0] - h2b
    r2 = r1 - c2b
    p22 = p12 * 1024 + b2

    scal_i3 = jnp.zeros((16,), jnp.int32).at[0:3].set(p22)
    hist3, mink = _pass3(parameters, scal_i3)
    h3 = jnp.sum(hist3, axis=0).reshape(3, L2_BINS)
    cum3 = jnp.cumsum(h3, axis=1)
    b3 = _first_ge(cum3, (r2 + 1)[:, None])
    c3 = jnp.take_along_axis(cum3, b3[:, None], axis=1)[:, 0]
    K = p22 * 1024 + b3
    v_lo = _unkey(K)

    cnt_le = cb1 + c2b + c3
    need_next = cnt_le < ranks + 2
    iota3 = lax.broadcasted_iota(jnp.int32, (3, L2_BINS), 1)
    above = (iota3 > b3[:, None]) & (h3 > 0)
    bnext = jnp.min(jnp.where(above, iota3, jnp.int32(L2_BINS)), axis=1)
    mk = jnp.min(mink.reshape(NW, 3, L), axis=(0, 2))
    Kn = jnp.where(bnext < L2_BINS, p22 * 1024 + bnext, mk)
    v_hi = jnp.where(need_next, _unkey(Kn), v_lo)

    qv = v_lo * jnp.asarray(LOW) + v_hi * jnp.asarray(HIW)

    scal = jnp.zeros((1, 128), jnp.float32)
    scal = scal.at[0, 0].set(mean).at[0, 1].set(std).at[0, 2].set(mx)
    scal = scal.at[0, 3].set(mn).at[0, 4].set(qv[0]).at[0, 5].set(qv[1])
    scal = scal.at[0, 6].set(qv[2])

    ecnt = jnp.sum(ehist, axis=0).astype(jnp.float32)[None, :]

    dens, fish, nent = _heads(
        ecnt, scal,
        dW1, db1[None, :], dg[None, :], dbeta[None, :], dW2, db2[None, :],
        fW1, fb1[None, :], fg[None, :], fbeta[None, :], fW2, fb2[None, :])
    return (dens.reshape(()), fish.reshape(()), nent.reshape(()))


# speedup vs baseline: 25.4942x; 1.0248x over previous
import functools

import numpy as np
import jax
import jax.numpy as jnp
from jax import lax
from jax.experimental import pallas as pl
from jax.experimental.pallas import tpu as pltpu
from jax.experimental.pallas import tpu_sc as plsc

N = 8388608
NC, NS, L = 2, 16, 16
NW = NC * NS
CHUNK = N // NW
BLK = 16384
NBLK = CHUNK // BLK
VPB = BLK // L

L1_BINS = 4096
L2_BINS = 1024
MASK31 = np.int32(0x7FFFFFFF)
I32MAX = np.int32(2147483647)

_QF = np.float32([0.1, 0.5, 0.9])
_QIDX = (_QF * (np.float32(N) - np.float32(1.0))).astype(np.float32)
RANK_LO = [int(x) for x in np.floor(_QIDX)]
RANK_HI = [int(x) for x in np.ceil(_QIDX)]
HIW = (_QIDX - np.floor(_QIDX)).astype(np.float32)
LOW = (np.float32(1.0) - HIW).astype(np.float32)

_mesh = plsc.VectorSubcoreMesh(core_axis_name="c", subcore_axis_name="s")


def _wid():
    return lax.axis_index("s") * NC + lax.axis_index("c")


def _keys(x):
    b = plsc.bitcast(x, jnp.int32)
    return jnp.where(b < 0, b ^ MASK31, b)


@functools.partial(
    pl.kernel,
    out_type=(
        jax.ShapeDtypeStruct((NW, L1_BINS * L), jnp.int32),
        jax.ShapeDtypeStruct((NW, 64), jnp.float32),
    ),
    mesh=_mesh,
    compiler_params=pltpu.CompilerParams(needs_layout_passes=False),
    scratch_types=[
        pltpu.VMEM((BLK,), jnp.float32),
        pltpu.VMEM((BLK,), jnp.float32),
        pltpu.VMEM((L1_BINS * L,), jnp.int32),
        pltpu.VMEM((64,), jnp.float32),
        pltpu.SemaphoreType.DMA,
        pltpu.SemaphoreType.DMA,
    ],
)
def _pass1(params, hist_out, mom_out, buf0, buf1, hist, momv, sem0, sem1):
    wid = _wid()
    base = wid * CHUNK
    lane = lax.iota(jnp.int32, L)
    lane_c = lane + jnp.int32(2048 * L)
    hi_mask = jnp.int32(-16)
    ones = jnp.ones((L,), jnp.int32)
    zeros16 = jnp.zeros((L,), jnp.int32)

    def zbody(i, _):
        for u in range(8):
            hist[pl.ds((i * 8 + u) * L, L)] = zeros16
        return 0
    lax.fori_loop(0, L1_BINS * L // (L * 8), zbody, 0)

    bufs = (buf0, buf1)
    sems = (sem0, sem1)
    cps = [None, None]
    cps[0] = pltpu.async_copy(params.at[pl.ds(base, BLK)], buf0, sem0)

    U = 4

    def make_body(buf):
        def body(i, carry):
            mnv0, mxv0, smv0, sqv0, mnv1, mxv1, smv1, sqv1 = carry
            accs = [[mnv0, mxv0, smv0, sqv0], [mnv1, mxv1, smv1, sqv1]]
            for u in range(U):
                x = buf[pl.ds((i * U + u) * L, L)]
                key = _keys(x)
                idx = ((key >> 16) & hi_mask) + lane_c
                plsc.addupdate_scatter(hist, [idx], ones)
                a = accs[u % 2]
                a[0] = jnp.minimum(a[0], x)
                a[1] = jnp.maximum(a[1], x)
                a[2] = a[2] + x
                a[3] = a[3] + x * x
            return tuple(accs[0]) + tuple(accs[1])
        return body

    inf = jnp.full((L,), jnp.inf, jnp.float32)
    zf = jnp.zeros((L,), jnp.float32)
    carry = (inf, -inf, zf, zf, inf, -inf, zf, zf)
    for bkt in range(NBLK):
        cur = bkt % 2
        if bkt + 1 < NBLK:
            cps[1 - cur] = pltpu.async_copy(
                params.at[pl.ds(base + (bkt + 1) * BLK, BLK)],
                bufs[1 - cur], sems[1 - cur])
        cps[cur].wait()
        carry = lax.fori_loop(0, VPB // U, make_body(bufs[cur]), carry)

    momv[pl.ds(0, L)] = jnp.minimum(carry[0], carry[4])
    momv[pl.ds(L, L)] = jnp.maximum(carry[1], carry[5])
    momv[pl.ds(2 * L, L)] = carry[2] + carry[6]
    momv[pl.ds(3 * L, L)] = carry[3] + carry[7]
    pltpu.sync_copy(momv, mom_out.at[wid])
    pltpu.sync_copy(hist, hist_out.at[wid])


@functools.partial(
    pl.kernel,
    out_type=(
        jax.ShapeDtypeStruct((NW, 3 * L2_BINS * L), jnp.int32),
        jax.ShapeDtypeStruct((NW, 16 * L), jnp.int32),
    ),
    mesh=_mesh,
    compiler_params=pltpu.CompilerParams(needs_layout_passes=False),
    scratch_types=[
        pltpu.VMEM((BLK,), jnp.float32),
        pltpu.VMEM((BLK,), jnp.float32),
        pltpu.VMEM((3 * L2_BINS * L,), jnp.int32),
        pltpu.VMEM((16 * L,), jnp.int32),
        pltpu.VMEM((16,), jnp.int32),
        pltpu.VMEM((16,), jnp.float32),
        pltpu.SemaphoreType.DMA,
        pltpu.SemaphoreType.DMA,
    ],
)
def _pass2(params, scal_i, scal_f, hist_out, ehist_out,
           buf0, buf1, chist, ehist, sbi, sbf, sem0, sem1):
    wid = _wid()
    base = wid * CHUNK
    lane = lax.iota(jnp.int32, L)
    ones = jnp.ones((L,), jnp.int32)
    zeros16 = jnp.zeros((L,), jnp.int32)

    pltpu.sync_copy(scal_i, sbi)
    pltpu.sync_copy(scal_f, sbf)
    sv_i = sbi[pl.ds(0, L)]
    sv_f = sbf[pl.ds(0, L)]
    p12_0 = sv_i[0]
    p12_1 = sv_i[1]
    p12_2 = sv_i[2]
    mn = sv_f[0]
    inv_w = sv_f[1]

    def zbody(i, _):
        for u in range(8):
            chist[pl.ds((i * 8 + u) * L, L)] = zeros16
        return 0
    lax.fori_loop(0, 3 * L2_BINS * L // (L * 8), zbody, 0)
    def zebody(i, _):
        ehist[pl.ds(i * L, L)] = zeros16
        return 0
    lax.fori_loop(0, 16, zebody, 0)

    bufs = (buf0, buf1)
    sems = (sem0, sem1)
    cps = [None, None]
    cps[0] = pltpu.async_copy(params.at[pl.ds(base, BLK)], buf0, sem0)

    bin_mask = jnp.int32(1023 * L)
    U = 4

    def make_body(buf):
        def body(i, _):
            for u in range(U):
                x = buf[pl.ds((i * U + u) * L, L)]
                key = _keys(x)
                hi12 = key >> 20
                idx0 = ((key >> 6) & bin_mask) + lane
                plsc.addupdate_scatter(chist, [idx0], ones,
                                       mask=hi12 == p12_0)
                plsc.addupdate_scatter(chist, [idx0 + jnp.int32(L2_BINS * L)],
                                       ones, mask=hi12 == p12_1)
                plsc.addupdate_scatter(chist,
                                       [idx0 + jnp.int32(2 * L2_BINS * L)],
                                       ones, mask=hi12 == p12_2)
                t = (x - mn) * inv_w
                ie = jnp.clip(t.astype(jnp.int32), 0, 15)
                plsc.addupdate_scatter(ehist, [(ie << 4) + lane], ones)
            return 0
        return body

    for bkt in range(NBLK):
        cur = bkt % 2
        if bkt + 1 < NBLK:
            cps[1 - cur] = pltpu.async_copy(
                params.at[pl.ds(base + (bkt + 1) * BLK, BLK)],
                bufs[1 - cur], sems[1 - cur])
        cps[cur].wait()
        lax.fori_loop(0, VPB // U, make_body(bufs[cur]), 0)

    pltpu.sync_copy(ehist, ehist_out.at[wid])
    pltpu.sync_copy(chist, hist_out.at[wid])


@functools.partial(
    pl.kernel,
    out_type=(
        jax.ShapeDtypeStruct((NW, 3 * L2_BINS * L), jnp.int32),
        jax.ShapeDtypeStruct((NW, 48), jnp.int32),
    ),
    mesh=_mesh,
    compiler_params=pltpu.CompilerParams(needs_layout_passes=False),
    scratch_types=[
        pltpu.VMEM((BLK,), jnp.float32),
        pltpu.VMEM((BLK,), jnp.float32),
        pltpu.VMEM((3 * L2_BINS * L,), jnp.int32),
        pltpu.VMEM((48,), jnp.int32),
        pltpu.VMEM((16,), jnp.int32),
        pltpu.SemaphoreType.DMA,
        pltpu.SemaphoreType.DMA,
    ],
)
def _pass3(params, scal_i, hist_out, mink_out,
           buf0, buf1, chist, minkv, sbi, sem0, sem1):
    wid = _wid()
    base = wid * CHUNK
    lane = lax.iota(jnp.int32, L)
    ones = jnp.ones((L,), jnp.int32)
    zeros16 = jnp.zeros((L,), jnp.int32)

    pltpu.sync_copy(scal_i, sbi)
    sv_i = sbi[pl.ds(0, L)]
    p22_0 = sv_i[0]
    p22_1 = sv_i[1]
    p22_2 = sv_i[2]

    def zbody(i, _):
        for u in range(8):
            chist[pl.ds((i * 8 + u) * L, L)] = zeros16
        return 0
    lax.fori_loop(0, 3 * L2_BINS * L // (L * 8), zbody, 0)

    bufs = (buf0, buf1)
    sems = (sem0, sem1)
    cps = [None, None]
    cps[0] = pltpu.async_copy(params.at[pl.ds(base, BLK)], buf0, sem0)

    bin_mask = jnp.int32(1023)
    U = 4

    def make_body(buf):
        def body(i, carry):
            mk0, mk1, mk2, mk3, mk4, mk5 = carry
            mks = [[mk0, mk1, mk2], [mk3, mk4, mk5]]
            for u in range(U):
                x = buf[pl.ds((i * U + u) * L, L)]
                key = _keys(x)
                hi22 = key >> 10
                idx0 = ((key & bin_mask) << 4) + lane
                plsc.addupdate_scatter(chist, [idx0], ones,
                                       mask=hi22 == p22_0)
                plsc.addupdate_scatter(chist, [idx0 + jnp.int32(L2_BINS * L)],
                                       ones, mask=hi22 == p22_1)
                plsc.addupdate_scatter(chist,
                                       [idx0 + jnp.int32(2 * L2_BINS * L)],
                                       ones, mask=hi22 == p22_2)
                mk = mks[u % 2]
                mk[0] = jnp.minimum(mk[0], jnp.where(hi22 > p22_0, key, I32MAX))
                mk[1] = jnp.minimum(mk[1], jnp.where(hi22 > p22_1, key, I32MAX))
                mk[2] = jnp.minimum(mk[2], jnp.where(hi22 > p22_2, key, I32MAX))
            return tuple(mks[0]) + tuple(mks[1])
        return body

    carry = (jnp.full((L,), I32MAX, jnp.int32),) * 6
    for bkt in range(NBLK):
        cur = bkt % 2
        if bkt + 1 < NBLK:
            cps[1 - cur] = pltpu.async_copy(
                params.at[pl.ds(base + (bkt + 1) * BLK, BLK)],
                bufs[1 - cur], sems[1 - cur])
        cps[cur].wait()
        carry = lax.fori_loop(0, VPB // U, make_body(bufs[cur]), carry)

    minkv[pl.ds(0, L)] = jnp.minimum(carry[0], carry[3])
    minkv[pl.ds(L, L)] = jnp.minimum(carry[1], carry[4])
    minkv[pl.ds(2 * L, L)] = jnp.minimum(carry[2], carry[5])
    pltpu.sync_copy(minkv, mink_out.at[wid])
    pltpu.sync_copy(chist, hist_out.at[wid])


def _heads_body(ecnt_ref, scal_ref,
                dW1_ref, db1_ref, dg_ref, dbeta_ref, dW2_ref, db2_ref,
                fW1_ref, fb1_ref, fg_ref, fbeta_ref, fW2_ref, fb2_ref,
                dens_ref, fish_ref, ent_ref):
    counts = ecnt_ref[0:1, :]
    probs = counts * jnp.float32(1.0 / N)
    logp = jnp.log(jnp.where(probs > 0, probs, jnp.float32(1.0)))
    ent = -jnp.sum(jnp.where(probs > 0, probs * logp, jnp.float32(0.0)))
    norm_ent = ent * jnp.float32(1.4426950408889634 / 4.0)
    ent_ref[...] = jnp.reshape(norm_ent, (1, 1))

    lanes = lax.broadcasted_iota(jnp.int32, (8, 128), 1)
    feat_f = jnp.broadcast_to(scal_ref[0:1, :], (8, 128))
    feat_d = jnp.where(lanes == 7, norm_ent, feat_f)

    def head(feat, W1, b1, g, beta, W2, b2):
        h = lax.dot_general(feat, W1[...], (((1,), (1,)), ((), ())),
                            preferred_element_type=jnp.float32)
        h = h + b1[0:1, :]
        mu = jnp.mean(h, axis=-1, keepdims=True)
        var = jnp.mean((h - mu) ** 2, axis=-1, keepdims=True)
        h = (h - mu) / jnp.sqrt(var + jnp.float32(1e-5)) * g[0:1, :] + beta[0:1, :]
        h = jnp.float32(0.5) * h * (jnp.float32(1.0) +
                                    lax.erf(h * jnp.float32(0.7071067811865476)))
        o = lax.dot_general(h, W2[...], (((1,), (1,)), ((), ())),
                            preferred_element_type=jnp.float32)
        return o[0:1, 0:1] + b2[0:1, 0:1]

    of = head(feat_f, fW1_ref, fb1_ref, fg_ref, fbeta_ref, fW2_ref, fb2_ref)
    fish_ref[...] = (jnp.maximum(of, 0.0) +
                     jnp.log1p(jnp.exp(-jnp.abs(of))))

    od = head(feat_d, dW1_ref, db1_ref, dg_ref, dbeta_ref, dW2_ref, db2_ref)
    dens_ref[...] = jnp.float32(1.0) / (jnp.float32(1.0) + jnp.exp(-od))


_heads = pl.pallas_call(
    _heads_body,
    out_shape=(
        jax.ShapeDtypeStruct((1, 1), jnp.float32),
        jax.ShapeDtypeStruct((1, 1), jnp.float32),
        jax.ShapeDtypeStruct((1, 1), jnp.float32),
    ),
)


def _first_ge(cum, target):
    return jnp.sum((cum < target).astype(jnp.int32), axis=-1)


def _unkey(k):
    return lax.bitcast_convert_type(jnp.where(k < 0, k ^ MASK31, k),
                                    jnp.float32)


def kernel(parameters, dW1, db1, dg, dbeta, dW2, db2,
           fW1, fb1, fg, fbeta, fW2, fb2):
    ranks = jnp.array(RANK_LO, jnp.int32)

    hist1, mom = _pass1(parameters)
    mom = mom.reshape(NW, 4, L)
    mn = jnp.min(mom[:, 0, :])
    mx = jnp.max(mom[:, 1, :])
    sm = jnp.sum(mom[:, 2, :])
    sq = jnp.sum(mom[:, 3, :])
    mean = sm / jnp.float32(N)
    var = (sq - sm * sm / jnp.float32(N)) / jnp.float32(N - 1)
    std = jnp.sqrt(var)

    h1 = jnp.sum(hist1.reshape(NW, L1_BINS, L), axis=(0, 2))
    cum1 = jnp.cumsum(h1)
    b1 = _first_ge(cum1[None, :], (ranks + 1)[:, None])
    cb1 = cum1[b1] - h1[b1]
    r1 = ranks - cb1
    p12 = b1 - 2048

    scal_i2 = jnp.zeros((16,), jnp.int32).at[0:3].set(p12)
    inv_w = jnp.float32(16.0) / (mx - mn)
    scal_f2 = jnp.zeros((16,), jnp.float32).at[0].set(mn).at[1].set(inv_w)

    hist2, ehist = _pass2(parameters, scal_i2, scal_f2)
    h2 = jnp.sum(hist2.reshape(NW, 3, L2_BINS, L), axis=(0, 3))
    cum2 = jnp.cumsum(h2, axis=1)
    b2 = _first_ge(cum2, (r1 + 1)[:, None])
    h2b = jnp.take_along_axis(h2, b2[:, None], axis=1)[:, 0]
    c2b = jnp.take_along_axis(cum2, b2[:, None], axis=1)[:, 0] - h2b
    r2 = r1 - c2b
    p22 = p12 * 1024 + b2

    scal_i3 = jnp.zeros((16,), jnp.int32).at[0:3].set(p22)
    hist3, mink = _pass3(parameters, scal_i3)
    h3 = jnp.sum(hist3.reshape(NW, 3, L2_BINS, L), axis=(0, 3))
    cum3 = jnp.cumsum(h3, axis=1)
    b3 = _first_ge(cum3, (r2 + 1)[:, None])
    c3 = jnp.take_along_axis(cum3, b3[:, None], axis=1)[:, 0]
    K = p22 * 1024 + b3
    v_lo = _unkey(K)

    cnt_le = cb1 + c2b + c3
    need_next = cnt_le < ranks + 2
    iota3 = lax.broadcasted_iota(jnp.int32, (3, L2_BINS), 1)
    above = (iota3 > b3[:, None]) & (h3 > 0)
    bnext = jnp.min(jnp.where(above, iota3, jnp.int32(L2_BINS)), axis=1)
    mk = jnp.min(mink.reshape(NW, 3, L), axis=(0, 2))
    Kn = jnp.where(bnext < L2_BINS, p22 * 1024 + bnext, mk)
    v_hi = jnp.where(need_next, _unkey(Kn), v_lo)

    qv = v_lo * jnp.asarray(LOW) + v_hi * jnp.asarray(HIW)

    scal = jnp.zeros((1, 128), jnp.float32)
    scal = scal.at[0, 0].set(mean).at[0, 1].set(std).at[0, 2].set(mx)
    scal = scal.at[0, 3].set(mn).at[0, 4].set(qv[0]).at[0, 5].set(qv[1])
    scal = scal.at[0, 6].set(qv[2])

    ecnt = jnp.sum(ehist.reshape(NW, 16, L), axis=(0, 2)).astype(jnp.float32)[None, :]

    dens, fish, nent = _heads(
        ecnt, scal,
        dW1, db1[None, :], dg[None, :], dbeta[None, :], dW2, db2[None, :],
        fW1, fb1[None, :], fg[None, :], fbeta[None, :], fW2, fb2[None, :])
    return (dens.reshape(()), fish.reshape(()), nent.reshape(()))


# speedup vs baseline: 45.7620x; 1.7950x over previous
import functools

import numpy as np
import jax
import jax.numpy as jnp
from jax import lax
from jax.experimental import pallas as pl
from jax.experimental.pallas import tpu as pltpu
from jax.experimental.pallas import tpu_sc as plsc

N = 8388608
NC, NS, L = 2, 16, 16
NW = NC * NS
CHUNK = N // NW
BLK = 16384
NBLK = CHUNK // BLK
VPB = BLK // L

L1_BINS = 4096
L2_BINS = 1024
MASK31 = np.int32(0x7FFFFFFF)
I32MAX = np.int32(2147483647)

_QF = np.float32([0.1, 0.5, 0.9])
_QIDX = (_QF * (np.float32(N) - np.float32(1.0))).astype(np.float32)
RANK_LO = [int(x) for x in np.floor(_QIDX)]
RANK_HI = [int(x) for x in np.ceil(_QIDX)]
HIW = (_QIDX - np.floor(_QIDX)).astype(np.float32)
LOW = (np.float32(1.0) - HIW).astype(np.float32)

_mesh = plsc.VectorSubcoreMesh(core_axis_name="c", subcore_axis_name="s")


def _wid():
    return lax.axis_index("s") * NC + lax.axis_index("c")


def _keys(x):
    b = plsc.bitcast(x, jnp.int32)
    return jnp.where(b < 0, b ^ MASK31, b)


@functools.partial(
    pl.kernel,
    out_type=(
        jax.ShapeDtypeStruct((NW, L1_BINS * L), jnp.int32),
        jax.ShapeDtypeStruct((NW, 64), jnp.float32),
    ),
    mesh=_mesh,
    compiler_params=pltpu.CompilerParams(needs_layout_passes=False),
    scratch_types=[
        pltpu.VMEM((BLK,), jnp.float32),
        pltpu.VMEM((BLK,), jnp.float32),
        pltpu.VMEM((L1_BINS * L,), jnp.int32),
        pltpu.VMEM((64,), jnp.float32),
        pltpu.SemaphoreType.DMA,
        pltpu.SemaphoreType.DMA,
    ],
)
def _pass1(params, hist_out, mom_out, buf0, buf1, hist, momv, sem0, sem1):
    wid = _wid()
    base = wid * CHUNK
    lane = lax.iota(jnp.int32, L)
    lane_c = lane + jnp.int32(2048 * L)
    hi_mask = jnp.int32(-16)
    ones = jnp.ones((L,), jnp.int32)
    zeros16 = jnp.zeros((L,), jnp.int32)

    def zbody(i, _):
        for u in range(8):
            hist[pl.ds((i * 8 + u) * L, L)] = zeros16
        return 0
    lax.fori_loop(0, L1_BINS * L // (L * 8), zbody, 0)

    bufs = (buf0, buf1)
    sems = (sem0, sem1)
    cps = [None, None]
    cps[0] = pltpu.async_copy(params.at[pl.ds(base, BLK)], buf0, sem0)

    def make_body(buf):
        def body(i, carry):
            accs = [list(carry[:4]), list(carry[4:])]
            for u in range(2):
                x = buf[pl.ds((i + u) * L, L)]
                key = _keys(x)
                idx = ((key >> 16) & hi_mask) + lane_c
                plsc.addupdate_scatter(hist, [idx], ones)
                a = accs[u]
                a[0] = jnp.minimum(a[0], x)
                a[1] = jnp.maximum(a[1], x)
                a[2] = a[2] + x
                a[3] = a[3] + x * x
            return tuple(accs[0]) + tuple(accs[1])
        return body

    inf = jnp.full((L,), jnp.inf, jnp.float32)
    zf = jnp.zeros((L,), jnp.float32)
    carry = (inf, -inf, zf, zf, inf, -inf, zf, zf)
    for bkt in range(NBLK):
        cur = bkt % 2
        if bkt + 1 < NBLK:
            cps[1 - cur] = pltpu.async_copy(
                params.at[pl.ds(base + (bkt + 1) * BLK, BLK)],
                bufs[1 - cur], sems[1 - cur])
        cps[cur].wait()
        carry = plsc.parallel_loop(0, VPB, step=2, unroll=4,
                                   carry=carry)(make_body(bufs[cur]))

    momv[pl.ds(0, L)] = jnp.minimum(carry[0], carry[4])
    momv[pl.ds(L, L)] = jnp.maximum(carry[1], carry[5])
    momv[pl.ds(2 * L, L)] = carry[2] + carry[6]
    momv[pl.ds(3 * L, L)] = carry[3] + carry[7]
    pltpu.sync_copy(momv, mom_out.at[wid])
    pltpu.sync_copy(hist, hist_out.at[wid])


@functools.partial(
    pl.kernel,
    out_type=(
        jax.ShapeDtypeStruct((NW, 3 * L2_BINS * L), jnp.int32),
        jax.ShapeDtypeStruct((NW, 16 * L), jnp.int32),
    ),
    mesh=_mesh,
    compiler_params=pltpu.CompilerParams(needs_layout_passes=False),
    scratch_types=[
        pltpu.VMEM((BLK,), jnp.float32),
        pltpu.VMEM((BLK,), jnp.float32),
        pltpu.VMEM((3 * L2_BINS * L,), jnp.int32),
        pltpu.VMEM((16 * L,), jnp.int32),
        pltpu.VMEM((16,), jnp.int32),
        pltpu.VMEM((16,), jnp.float32),
        pltpu.SemaphoreType.DMA,
        pltpu.SemaphoreType.DMA,
    ],
)
def _pass2(params, scal_i, scal_f, hist_out, ehist_out,
           buf0, buf1, chist, ehist, sbi, sbf, sem0, sem1):
    wid = _wid()
    base = wid * CHUNK
    lane = lax.iota(jnp.int32, L)
    ones = jnp.ones((L,), jnp.int32)
    zeros16 = jnp.zeros((L,), jnp.int32)

    pltpu.sync_copy(scal_i, sbi)
    pltpu.sync_copy(scal_f, sbf)
    sv_i = sbi[pl.ds(0, L)]
    sv_f = sbf[pl.ds(0, L)]
    p12_0 = sv_i[0]
    p12_1 = sv_i[1]
    p12_2 = sv_i[2]
    mn = sv_f[0]
    inv_w = sv_f[1]

    def zbody(i, _):
        for u in range(8):
            chist[pl.ds((i * 8 + u) * L, L)] = zeros16
        return 0
    lax.fori_loop(0, 3 * L2_BINS * L // (L * 8), zbody, 0)
    def zebody(i, _):
        ehist[pl.ds(i * L, L)] = zeros16
        return 0
    lax.fori_loop(0, 16, zebody, 0)

    bufs = (buf0, buf1)
    sems = (sem0, sem1)
    cps = [None, None]
    cps[0] = pltpu.async_copy(params.at[pl.ds(base, BLK)], buf0, sem0)

    bin_mask = jnp.int32(1023 * L)

    def make_body(buf):
        def body(i, c):
            x = buf[pl.ds(i * L, L)]
            key = _keys(x)
            hi12 = key >> 20
            idx0 = ((key >> 6) & bin_mask) + lane
            plsc.addupdate_scatter(chist, [idx0], ones,
                                   mask=hi12 == p12_0)
            plsc.addupdate_scatter(chist, [idx0 + jnp.int32(L2_BINS * L)],
                                   ones, mask=hi12 == p12_1)
            plsc.addupdate_scatter(chist,
                                   [idx0 + jnp.int32(2 * L2_BINS * L)],
                                   ones, mask=hi12 == p12_2)
            t = (x - mn) * inv_w
            ie = jnp.clip(t.astype(jnp.int32), 0, 15)
            plsc.addupdate_scatter(ehist, [(ie << 4) + lane], ones)
            return c
        return body

    for bkt in range(NBLK):
        cur = bkt % 2
        if bkt + 1 < NBLK:
            cps[1 - cur] = pltpu.async_copy(
                params.at[pl.ds(base + (bkt + 1) * BLK, BLK)],
                bufs[1 - cur], sems[1 - cur])
        cps[cur].wait()
        plsc.parallel_loop(0, VPB, step=1, unroll=8,
                           carry=jnp.int32(0))(make_body(bufs[cur]))

    pltpu.sync_copy(ehist, ehist_out.at[wid])
    pltpu.sync_copy(chist, hist_out.at[wid])


@functools.partial(
    pl.kernel,
    out_type=(
        jax.ShapeDtypeStruct((NW, 3 * L2_BINS * L), jnp.int32),
        jax.ShapeDtypeStruct((NW, 48), jnp.int32),
    ),
    mesh=_mesh,
    compiler_params=pltpu.CompilerParams(needs_layout_passes=False),
    scratch_types=[
        pltpu.VMEM((BLK,), jnp.float32),
        pltpu.VMEM((BLK,), jnp.float32),
        pltpu.VMEM((3 * L2_BINS * L,), jnp.int32),
        pltpu.VMEM((48,), jnp.int32),
        pltpu.VMEM((16,), jnp.int32),
        pltpu.SemaphoreType.DMA,
        pltpu.SemaphoreType.DMA,
    ],
)
def _pass3(params, scal_i, hist_out, mink_out,
           buf0, buf1, chist, minkv, sbi, sem0, sem1):
    wid = _wid()
    base = wid * CHUNK
    lane = lax.iota(jnp.int32, L)
    ones = jnp.ones((L,), jnp.int32)
    zeros16 = jnp.zeros((L,), jnp.int32)

    pltpu.sync_copy(scal_i, sbi)
    sv_i = sbi[pl.ds(0, L)]
    p22_0 = sv_i[0]
    p22_1 = sv_i[1]
    p22_2 = sv_i[2]

    def zbody(i, _):
        for u in range(8):
            chist[pl.ds((i * 8 + u) * L, L)] = zeros16
        return 0
    lax.fori_loop(0, 3 * L2_BINS * L // (L * 8), zbody, 0)

    bufs = (buf0, buf1)
    sems = (sem0, sem1)
    cps = [None, None]
    cps[0] = pltpu.async_copy(params.at[pl.ds(base, BLK)], buf0, sem0)

    bin_mask = jnp.int32(1023)

    def make_body(buf):
        def body(i, carry):
            mks = [list(carry[:3]), list(carry[3:])]
            for u in range(2):
                x = buf[pl.ds((i + u) * L, L)]
                key = _keys(x)
                hi22 = key >> 10
                idx0 = ((key & bin_mask) << 4) + lane
                plsc.addupdate_scatter(chist, [idx0], ones,
                                       mask=hi22 == p22_0)
                plsc.addupdate_scatter(chist, [idx0 + jnp.int32(L2_BINS * L)],
                                       ones, mask=hi22 == p22_1)
                plsc.addupdate_scatter(chist,
                                       [idx0 + jnp.int32(2 * L2_BINS * L)],
                                       ones, mask=hi22 == p22_2)
                mk = mks[u]
                mk[0] = jnp.minimum(mk[0], jnp.where(hi22 > p22_0, key, I32MAX))
                mk[1] = jnp.minimum(mk[1], jnp.where(hi22 > p22_1, key, I32MAX))
                mk[2] = jnp.minimum(mk[2], jnp.where(hi22 > p22_2, key, I32MAX))
            return tuple(mks[0]) + tuple(mks[1])
        return body

    carry = (jnp.full((L,), I32MAX, jnp.int32),) * 6
    for bkt in range(NBLK):
        cur = bkt % 2
        if bkt + 1 < NBLK:
            cps[1 - cur] = pltpu.async_copy(
                params.at[pl.ds(base + (bkt + 1) * BLK, BLK)],
                bufs[1 - cur], sems[1 - cur])
        cps[cur].wait()
        carry = plsc.parallel_loop(0, VPB, step=2, unroll=4,
                                   carry=carry)(make_body(bufs[cur]))

    minkv[pl.ds(0, L)] = jnp.minimum(carry[0], carry[3])
    minkv[pl.ds(L, L)] = jnp.minimum(carry[1], carry[4])
    minkv[pl.ds(2 * L, L)] = jnp.minimum(carry[2], carry[5])
    pltpu.sync_copy(minkv, mink_out.at[wid])
    pltpu.sync_copy(chist, hist_out.at[wid])


def _heads_body(ecnt_ref, scal_ref,
                dW1_ref, db1_ref, dg_ref, dbeta_ref, dW2_ref, db2_ref,
                fW1_ref, fb1_ref, fg_ref, fbeta_ref, fW2_ref, fb2_ref,
                dens_ref, fish_ref, ent_ref):
    counts = ecnt_ref[0:1, :]
    probs = counts * jnp.float32(1.0 / N)
    logp = jnp.log(jnp.where(probs > 0, probs, jnp.float32(1.0)))
    ent = -jnp.sum(jnp.where(probs > 0, probs * logp, jnp.float32(0.0)))
    norm_ent = ent * jnp.float32(1.4426950408889634 / 4.0)
    ent_ref[...] = jnp.reshape(norm_ent, (1, 1))

    lanes = lax.broadcasted_iota(jnp.int32, (8, 128), 1)
    feat_f = jnp.broadcast_to(scal_ref[0:1, :], (8, 128))
    feat_d = jnp.where(lanes == 7, norm_ent, feat_f)

    def head(feat, W1, b1, g, beta, W2, b2):
        h = lax.dot_general(feat, W1[...], (((1,), (1,)), ((), ())),
                            preferred_element_type=jnp.float32)
        h = h + b1[0:1, :]
        mu = jnp.mean(h, axis=-1, keepdims=True)
        var = jnp.mean((h - mu) ** 2, axis=-1, keepdims=True)
        h = (h - mu) / jnp.sqrt(var + jnp.float32(1e-5)) * g[0:1, :] + beta[0:1, :]
        h = jnp.float32(0.5) * h * (jnp.float32(1.0) +
                                    lax.erf(h * jnp.float32(0.7071067811865476)))
        o = lax.dot_general(h, W2[...], (((1,), (1,)), ((), ())),
                            preferred_element_type=jnp.float32)
        return o[0:1, 0:1] + b2[0:1, 0:1]

    of = head(feat_f, fW1_ref, fb1_ref, fg_ref, fbeta_ref, fW2_ref, fb2_ref)
    fish_ref[...] = (jnp.maximum(of, 0.0) +
                     jnp.log1p(jnp.exp(-jnp.abs(of))))

    od = head(feat_d, dW1_ref, db1_ref, dg_ref, dbeta_ref, dW2_ref, db2_ref)
    dens_ref[...] = jnp.float32(1.0) / (jnp.float32(1.0) + jnp.exp(-od))


_heads = pl.pallas_call(
    _heads_body,
    out_shape=(
        jax.ShapeDtypeStruct((1, 1), jnp.float32),
        jax.ShapeDtypeStruct((1, 1), jnp.float32),
        jax.ShapeDtypeStruct((1, 1), jnp.float32),
    ),
)


def _first_ge(cum, target):
    return jnp.sum((cum < target).astype(jnp.int32), axis=-1)


def _unkey(k):
    return lax.bitcast_convert_type(jnp.where(k < 0, k ^ MASK31, k),
                                    jnp.float32)


def kernel(parameters, dW1, db1, dg, dbeta, dW2, db2,
           fW1, fb1, fg, fbeta, fW2, fb2):
    ranks = jnp.array(RANK_LO, jnp.int32)

    hist1, mom = _pass1(parameters)
    mom = mom.reshape(NW, 4, L)
    mn = jnp.min(mom[:, 0, :])
    mx = jnp.max(mom[:, 1, :])
    sm = jnp.sum(mom[:, 2, :])
    sq = jnp.sum(mom[:, 3, :])
    mean = sm / jnp.float32(N)
    var = (sq - sm * sm / jnp.float32(N)) / jnp.float32(N - 1)
    std = jnp.sqrt(var)

    h1 = jnp.sum(hist1.reshape(NW, L1_BINS, L), axis=(0, 2))
    cum1 = jnp.cumsum(h1)
    b1 = _first_ge(cum1[None, :], (ranks + 1)[:, None])
    cb1 = cum1[b1] - h1[b1]
    r1 = ranks - cb1
    p12 = b1 - 2048

    scal_i2 = jnp.zeros((16,), jnp.int32).at[0:3].set(p12)
    inv_w = jnp.float32(16.0) / (mx - mn)
    scal_f2 = jnp.zeros((16,), jnp.float32).at[0].set(mn).at[1].set(inv_w)

    hist2, ehist = _pass2(parameters, scal_i2, scal_f2)
    h2 = jnp.sum(hist2.reshape(NW, 3, L2_BINS, L), axis=(0, 3))
    cum2 = jnp.cumsum(h2, axis=1)
    b2 = _first_ge(cum2, (r1 + 1)[:, None])
    h2b = jnp.take_along_axis(h2, b2[:, None], axis=1)[:, 0]
    c2b = jnp.take_along_axis(cum2, b2[:, None], axis=1)[:, 0] - h2b
    r2 = r1 - c2b
    p22 = p12 * 1024 + b2

    scal_i3 = jnp.zeros((16,), jnp.int32).at[0:3].set(p22)
    hist3, mink = _pass3(parameters, scal_i3)
    h3 = jnp.sum(hist3.reshape(NW, 3, L2_BINS, L), axis=(0, 3))
    cum3 = jnp.cumsum(h3, axis=1)
    b3 = _first_ge(cum3, (r2 + 1)[:, None])
    c3 = jnp.take_along_axis(cum3, b3[:, None], axis=1)[:, 0]
    K = p22 * 1024 + b3
    v_lo = _unkey(K)

    cnt_le = cb1 + c2b + c3
    need_next = cnt_le < ranks + 2
    iota3 = lax.broadcasted_iota(jnp.int32, (3, L2_BINS), 1)
    above = (iota3 > b3[:, None]) & (h3 > 0)
    bnext = jnp.min(jnp.where(above, iota3, jnp.int32(L2_BINS)), axis=1)
    mk = jnp.min(mink.reshape(NW, 3, L), axis=(0, 2))
    Kn = jnp.where(bnext < L2_BINS, p22 * 1024 + bnext, mk)
    v_hi = jnp.where(need_next, _unkey(Kn), v_lo)

    qv = v_lo * jnp.asarray(LOW) + v_hi * jnp.asarray(HIW)

    scal = jnp.zeros((1, 128), jnp.float32)
    scal = scal.at[0, 0].set(mean).at[0, 1].set(std).at[0, 2].set(mx)
    scal = scal.at[0, 3].set(mn).at[0, 4].set(qv[0]).at[0, 5].set(qv[1])
    scal = scal.at[0, 6].set(qv[2])

    ecnt = jnp.sum(ehist.reshape(NW, 16, L), axis=(0, 2)).astype(jnp.float32)[None, :]

    dens, fish, nent = _heads(
        ecnt, scal,
        dW1, db1[None, :], dg[None, :], dbeta[None, :], dW2, db2[None, :],
        fW1, fb1[None, :], fg[None, :], fbeta[None, :], fW2, fb2[None, :])
    return (dens.reshape(()), fish.reshape(()), nent.reshape(()))


# speedup vs baseline: 54.6628x; 1.1945x over previous
import functools

import numpy as np
import jax
import jax.numpy as jnp
from jax import lax
from jax.experimental import pallas as pl
from jax.experimental.pallas import tpu as pltpu
from jax.experimental.pallas import tpu_sc as plsc

N = 8388608
NC, NS, L = 2, 16, 16
NW = NC * NS
CHUNK = N // NW
BLK = 16384
NBLK = CHUNK // BLK
VPB = BLK // L

L1_BINS = 4096
L2_BINS = 1024
MASK31 = np.int32(0x7FFFFFFF)
I32MAX = np.int32(2147483647)

_QF = np.float32([0.1, 0.5, 0.9])
_QIDX = (_QF * (np.float32(N) - np.float32(1.0))).astype(np.float32)
RANK_LO = [int(x) for x in np.floor(_QIDX)]
RANK_HI = [int(x) for x in np.ceil(_QIDX)]
HIW = (_QIDX - np.floor(_QIDX)).astype(np.float32)
LOW = (np.float32(1.0) - HIW).astype(np.float32)

_mesh = plsc.VectorSubcoreMesh(core_axis_name="c", subcore_axis_name="s")


def _wid():
    return lax.axis_index("s") * NC + lax.axis_index("c")


def _keys(x):
    b = plsc.bitcast(x, jnp.int32)
    return jnp.where(b < 0, b ^ MASK31, b)


@functools.partial(
    pl.kernel,
    out_type=(
        jax.ShapeDtypeStruct((NW, L1_BINS * L), jnp.int32),
        jax.ShapeDtypeStruct((NW, 64), jnp.float32),
    ),
    mesh=_mesh,
    compiler_params=pltpu.CompilerParams(needs_layout_passes=False),
    scratch_types=[
        pltpu.VMEM((BLK,), jnp.float32),
        pltpu.VMEM((BLK,), jnp.float32),
        pltpu.VMEM((L1_BINS * L,), jnp.int32),
        pltpu.VMEM((64,), jnp.float32),
        pltpu.SemaphoreType.DMA,
        pltpu.SemaphoreType.DMA,
    ],
)
def _pass1(params, hist_out, mom_out, buf0, buf1, hist, momv, sem0, sem1):
    wid = _wid()
    base = wid * CHUNK
    lane = lax.iota(jnp.int32, L)
    lane_c = lane + jnp.int32(2048 * L)
    hi_mask = jnp.int32(-16)
    ones = jnp.ones((L,), jnp.int32)
    zeros16 = jnp.zeros((L,), jnp.int32)

    def zbody(i, _):
        for u in range(8):
            hist[pl.ds((i * 8 + u) * L, L)] = zeros16
        return 0
    lax.fori_loop(0, L1_BINS * L // (L * 8), zbody, 0)

    bufs = (buf0, buf1)
    sems = (sem0, sem1)
    cps = [None, None]
    cps[0] = pltpu.async_copy(params.at[pl.ds(base, BLK)], buf0, sem0)

    def make_body(buf):
        def body(i, carry):
            accs = [list(carry[:4]), list(carry[4:])]
            for u in range(2):
                x = buf[pl.ds((i + u) * L, L)]
                key = _keys(x)
                idx = ((key >> 16) & hi_mask) + lane_c
                plsc.addupdate_scatter(hist, [idx], ones)
                a = accs[u]
                a[0] = jnp.minimum(a[0], x)
                a[1] = jnp.maximum(a[1], x)
                a[2] = a[2] + x
                a[3] = a[3] + x * x
            return tuple(accs[0]) + tuple(accs[1])
        return body

    inf = jnp.full((L,), jnp.inf, jnp.float32)
    zf = jnp.zeros((L,), jnp.float32)
    carry = (inf, -inf, zf, zf, inf, -inf, zf, zf)
    for bkt in range(NBLK):
        cur = bkt % 2
        if bkt + 1 < NBLK:
            cps[1 - cur] = pltpu.async_copy(
                params.at[pl.ds(base + (bkt + 1) * BLK, BLK)],
                bufs[1 - cur], sems[1 - cur])
        cps[cur].wait()
        carry = plsc.parallel_loop(0, VPB, step=2, unroll=4,
                                   carry=carry)(make_body(bufs[cur]))

    momv[pl.ds(0, L)] = jnp.minimum(carry[0], carry[4])
    momv[pl.ds(L, L)] = jnp.maximum(carry[1], carry[5])
    momv[pl.ds(2 * L, L)] = carry[2] + carry[6]
    momv[pl.ds(3 * L, L)] = carry[3] + carry[7]
    pltpu.sync_copy(momv, mom_out.at[wid])
    pltpu.sync_copy(hist, hist_out.at[wid])


@functools.partial(
    pl.kernel,
    out_type=(
        jax.ShapeDtypeStruct((NW, 3 * L2_BINS * L), jnp.int32),
        jax.ShapeDtypeStruct((NW, 16 * L), jnp.int32),
    ),
    mesh=_mesh,
    compiler_params=pltpu.CompilerParams(needs_layout_passes=False),
    scratch_types=[
        pltpu.VMEM((BLK,), jnp.float32),
        pltpu.VMEM((BLK,), jnp.float32),
        pltpu.VMEM((3 * L2_BINS * L,), jnp.int32),
        pltpu.VMEM((16 * L,), jnp.int32),
        pltpu.VMEM((16,), jnp.int32),
        pltpu.VMEM((16,), jnp.float32),
        pltpu.SemaphoreType.DMA,
        pltpu.SemaphoreType.DMA,
    ],
)
def _pass2(params, scal_i, scal_f, hist_out, ehist_out,
           buf0, buf1, chist, ehist, sbi, sbf, sem0, sem1):
    wid = _wid()
    base = wid * CHUNK
    lane = lax.iota(jnp.int32, L)
    ones = jnp.ones((L,), jnp.int32)
    zeros16 = jnp.zeros((L,), jnp.int32)

    pltpu.sync_copy(scal_i, sbi)
    pltpu.sync_copy(scal_f, sbf)
    sv_i = sbi[pl.ds(0, L)]
    sv_f = sbf[pl.ds(0, L)]
    p12_0 = sv_i[0]
    p12_1 = sv_i[1]
    p12_2 = sv_i[2]
    mn = sv_f[0]
    inv_w = sv_f[1]

    def zbody(i, _):
        for u in range(8):
            chist[pl.ds((i * 8 + u) * L, L)] = zeros16
        return 0
    lax.fori_loop(0, 3 * L2_BINS * L // (L * 8), zbody, 0)
    def zebody(i, _):
        ehist[pl.ds(i * L, L)] = zeros16
        return 0
    lax.fori_loop(0, 16, zebody, 0)

    bufs = (buf0, buf1)
    sems = (sem0, sem1)
    cps = [None, None]
    cps[0] = pltpu.async_copy(params.at[pl.ds(base, BLK)], buf0, sem0)

    bin_mask = jnp.int32(1023 * L)

    def make_body(buf):
        def body(i, c):
            x = buf[pl.ds(i * L, L)]
            key = _keys(x)
            hi12 = key >> 20
            idx0 = ((key >> 6) & bin_mask) + lane
            plsc.addupdate_scatter(chist, [idx0], ones,
                                   mask=hi12 == p12_0)
            plsc.addupdate_scatter(chist, [idx0 + jnp.int32(L2_BINS * L)],
                                   ones, mask=hi12 == p12_1)
            plsc.addupdate_scatter(chist,
                                   [idx0 + jnp.int32(2 * L2_BINS * L)],
                                   ones, mask=hi12 == p12_2)
            t = (x - mn) * inv_w
            ie = jnp.clip(t.astype(jnp.int32), 0, 15)
            plsc.addupdate_scatter(ehist, [(ie << 4) + lane], ones)
            return c
        return body

    for bkt in range(NBLK):
        cur = bkt % 2
        if bkt + 1 < NBLK:
            cps[1 - cur] = pltpu.async_copy(
                params.at[pl.ds(base + (bkt + 1) * BLK, BLK)],
                bufs[1 - cur], sems[1 - cur])
        cps[cur].wait()
        plsc.parallel_loop(0, VPB, step=1, unroll=8,
                           carry=jnp.int32(0))(make_body(bufs[cur]))

    pltpu.sync_copy(ehist, ehist_out.at[wid])
    pltpu.sync_copy(chist, hist_out.at[wid])


@functools.partial(
    pl.kernel,
    out_type=(
        jax.ShapeDtypeStruct((NW, 3 * L2_BINS * L), jnp.int32),
        jax.ShapeDtypeStruct((NW, 48), jnp.int32),
    ),
    mesh=_mesh,
    compiler_params=pltpu.CompilerParams(needs_layout_passes=False),
    scratch_types=[
        pltpu.VMEM((BLK,), jnp.float32),
        pltpu.VMEM((BLK,), jnp.float32),
        pltpu.VMEM((3 * L2_BINS * L,), jnp.int32),
        pltpu.VMEM((48,), jnp.int32),
        pltpu.VMEM((16,), jnp.int32),
        pltpu.SemaphoreType.DMA,
        pltpu.SemaphoreType.DMA,
    ],
)
def _pass3(params, scal_i, hist_out, mink_out,
           buf0, buf1, chist, minkv, sbi, sem0, sem1):
    wid = _wid()
    base = wid * CHUNK
    lane = lax.iota(jnp.int32, L)
    ones = jnp.ones((L,), jnp.int32)
    zeros16 = jnp.zeros((L,), jnp.int32)

    pltpu.sync_copy(scal_i, sbi)
    sv_i = sbi[pl.ds(0, L)]
    p22_0 = sv_i[0]
    p22_1 = sv_i[1]
    p22_2 = sv_i[2]

    def zbody(i, _):
        for u in range(8):
            chist[pl.ds((i * 8 + u) * L, L)] = zeros16
        return 0
    lax.fori_loop(0, 3 * L2_BINS * L // (L * 8), zbody, 0)

    bufs = (buf0, buf1)
    sems = (sem0, sem1)
    cps = [None, None]
    cps[0] = pltpu.async_copy(params.at[pl.ds(base, BLK)], buf0, sem0)

    bin_mask = jnp.int32(1023)

    def make_body(buf):
        def body(i, carry):
            mks = [list(carry[:3]), list(carry[3:])]
            for u in range(2):
                x = buf[pl.ds((i + u) * L, L)]
                key = _keys(x)
                hi22 = key >> 10
                idx0 = ((key & bin_mask) << 4) + lane
                plsc.addupdate_scatter(chist, [idx0], ones,
                                       mask=hi22 == p22_0)
                plsc.addupdate_scatter(chist, [idx0 + jnp.int32(L2_BINS * L)],
                                       ones, mask=hi22 == p22_1)
                plsc.addupdate_scatter(chist,
                                       [idx0 + jnp.int32(2 * L2_BINS * L)],
                                       ones, mask=hi22 == p22_2)
                mk = mks[u]
                mk[0] = jnp.minimum(mk[0], jnp.where(hi22 > p22_0, key, I32MAX))
                mk[1] = jnp.minimum(mk[1], jnp.where(hi22 > p22_1, key, I32MAX))
                mk[2] = jnp.minimum(mk[2], jnp.where(hi22 > p22_2, key, I32MAX))
            return tuple(mks[0]) + tuple(mks[1])
        return body

    carry = (jnp.full((L,), I32MAX, jnp.int32),) * 6
    for bkt in range(NBLK):
        cur = bkt % 2
        if bkt + 1 < NBLK:
            cps[1 - cur] = pltpu.async_copy(
                params.at[pl.ds(base + (bkt + 1) * BLK, BLK)],
                bufs[1 - cur], sems[1 - cur])
        cps[cur].wait()
        carry = plsc.parallel_loop(0, VPB, step=2, unroll=2,
                                   carry=carry)(make_body(bufs[cur]))

    minkv[pl.ds(0, L)] = jnp.minimum(carry[0], carry[3])
    minkv[pl.ds(L, L)] = jnp.minimum(carry[1], carry[4])
    minkv[pl.ds(2 * L, L)] = jnp.minimum(carry[2], carry[5])
    pltpu.sync_copy(minkv, mink_out.at[wid])
    pltpu.sync_copy(chist, hist_out.at[wid])


def _heads_body(ecnt_ref, scal_ref,
                dW1_ref, db1_ref, dg_ref, dbeta_ref, dW2_ref, db2_ref,
                fW1_ref, fb1_ref, fg_ref, fbeta_ref, fW2_ref, fb2_ref,
                dens_ref, fish_ref, ent_ref):
    counts = ecnt_ref[0:1, :]
    probs = counts * jnp.float32(1.0 / N)
    logp = jnp.log(jnp.where(probs > 0, probs, jnp.float32(1.0)))
    ent = -jnp.sum(jnp.where(probs > 0, probs * logp, jnp.float32(0.0)))
    norm_ent = ent * jnp.float32(1.4426950408889634 / 4.0)
    ent_ref[...] = jnp.reshape(norm_ent, (1, 1))

    lanes = lax.broadcasted_iota(jnp.int32, (8, 128), 1)
    feat_f = jnp.broadcast_to(scal_ref[0:1, :], (8, 128))
    feat_d = jnp.where(lanes == 7, norm_ent, feat_f)

    def head(feat, W1, b1, g, beta, W2, b2):
        h = lax.dot_general(feat, W1[...], (((1,), (1,)), ((), ())),
                            preferred_element_type=jnp.float32)
        h = h + b1[0:1, :]
        mu = jnp.mean(h, axis=-1, keepdims=True)
        var = jnp.mean((h - mu) ** 2, axis=-1, keepdims=True)
        h = (h - mu) / jnp.sqrt(var + jnp.float32(1e-5)) * g[0:1, :] + beta[0:1, :]
        h = jnp.float32(0.5) * h * (jnp.float32(1.0) +
                                    lax.erf(h * jnp.float32(0.7071067811865476)))
        o = lax.dot_general(h, W2[...], (((1,), (1,)), ((), ())),
                            preferred_element_type=jnp.float32)
        return o[0:1, 0:1] + b2[0:1, 0:1]

    of = head(feat_f, fW1_ref, fb1_ref, fg_ref, fbeta_ref, fW2_ref, fb2_ref)
    fish_ref[...] = (jnp.maximum(of, 0.0) +
                     jnp.log1p(jnp.exp(-jnp.abs(of))))

    od = head(feat_d, dW1_ref, db1_ref, dg_ref, dbeta_ref, dW2_ref, db2_ref)
    dens_ref[...] = jnp.float32(1.0) / (jnp.float32(1.0) + jnp.exp(-od))


_heads = pl.pallas_call(
    _heads_body,
    out_shape=(
        jax.ShapeDtypeStruct((1, 1), jnp.float32),
        jax.ShapeDtypeStruct((1, 1), jnp.float32),
        jax.ShapeDtypeStruct((1, 1), jnp.float32),
    ),
)


def _first_ge(cum, target):
    return jnp.sum((cum < target).astype(jnp.int32), axis=-1)


def _unkey(k):
    return lax.bitcast_convert_type(jnp.where(k < 0, k ^ MASK31, k),
                                    jnp.float32)


def kernel(parameters, dW1, db1, dg, dbeta, dW2, db2,
           fW1, fb1, fg, fbeta, fW2, fb2):
    ranks = jnp.array(RANK_LO, jnp.int32)

    hist1, mom = _pass1(parameters)
    mom = mom.reshape(NW, 4, L)
    mn = jnp.min(mom[:, 0, :])
    mx = jnp.max(mom[:, 1, :])
    sm = jnp.sum(mom[:, 2, :])
    sq = jnp.sum(mom[:, 3, :])
    mean = sm / jnp.float32(N)
    var = (sq - sm * sm / jnp.float32(N)) / jnp.float32(N - 1)
    std = jnp.sqrt(var)

    h1 = jnp.sum(hist1.reshape(NW, L1_BINS, L), axis=(0, 2))
    cum1 = jnp.cumsum(h1)
    b1 = _first_ge(cum1[None, :], (ranks + 1)[:, None])
    cb1 = cum1[b1] - h1[b1]
    r1 = ranks - cb1
    p12 = b1 - 2048

    scal_i2 = jnp.zeros((16,), jnp.int32).at[0:3].set(p12)
    inv_w = jnp.float32(16.0) / (mx - mn)
    scal_f2 = jnp.zeros((16,), jnp.float32).at[0].set(mn).at[1].set(inv_w)

    hist2, ehist = _pass2(parameters, scal_i2, scal_f2)
    h2 = jnp.sum(hist2.reshape(NW, 3, L2_BINS, L), axis=(0, 3))
    cum2 = jnp.cumsum(h2, axis=1)
    b2 = _first_ge(cum2, (r1 + 1)[:, None])
    h2b = jnp.take_along_axis(h2, b2[:, None], axis=1)[:, 0]
    c2b = jnp.take_along_axis(cum2, b2[:, None], axis=1)[:, 0] - h2b
    r2 = r1 - c2b
    p22 = p12 * 1024 + b2

    scal_i3 = jnp.zeros((16,), jnp.int32).at[0:3].set(p22)
    hist3, mink = _pass3(parameters, scal_i3)
    h3 = jnp.sum(hist3.reshape(NW, 3, L2_BINS, L), axis=(0, 3))
    cum3 = jnp.cumsum(h3, axis=1)
    b3 = _first_ge(cum3, (r2 + 1)[:, None])
    c3 = jnp.take_along_axis(cum3, b3[:, None], axis=1)[:, 0]
    K = p22 * 1024 + b3
    v_lo = _unkey(K)

    cnt_le = cb1 + c2b + c3
    need_next = cnt_le < ranks + 2
    iota3 = lax.broadcasted_iota(jnp.int32, (3, L2_BINS), 1)
    above = (iota3 > b3[:, None]) & (h3 > 0)
    bnext = jnp.min(jnp.where(above, iota3, jnp.int32(L2_BINS)), axis=1)
    mk = jnp.min(mink.reshape(NW, 3, L), axis=(0, 2))
    Kn = jnp.where(bnext < L2_BINS, p22 * 1024 + bnext, mk)
    v_hi = jnp.where(need_next, _unkey(Kn), v_lo)

    qv = v_lo * jnp.asarray(LOW) + v_hi * jnp.asarray(HIW)

    scal = jnp.zeros((1, 128), jnp.float32)
    scal = scal.at[0, 0].set(mean).at[0, 1].set(std).at[0, 2].set(mx)
    scal = scal.at[0, 3].set(mn).at[0, 4].set(qv[0]).at[0, 5].set(qv[1])
    scal = scal.at[0, 6].set(qv[2])

    ecnt = jnp.sum(ehist.reshape(NW, 16, L), axis=(0, 2)).astype(jnp.float32)[None, :]

    dens, fish, nent = _heads(
        ecnt, scal,
        dW1, db1[None, :], dg[None, :], dbeta[None, :], dW2, db2[None, :],
        fW1, fb1[None, :], fg[None, :], fbeta[None, :], fW2, fb2[None, :])
    return (dens.reshape(()), fish.reshape(()), nent.reshape(()))


# speedup vs baseline: 54.8107x; 1.0027x over previous
import functools

import numpy as np
import jax
import jax.numpy as jnp
from jax import lax
from jax.experimental import pallas as pl
from jax.experimental.pallas import tpu as pltpu
from jax.experimental.pallas import tpu_sc as plsc

N = 8388608
NC, NS, L = 2, 16, 16
NW = NC * NS
CHUNK = N // NW
BLK = 16384
NBLK = CHUNK // BLK
VPB = BLK // L

L1_BINS = 4096
L2_BINS = 1024
MASK31 = np.int32(0x7FFFFFFF)
I32MAX = np.int32(2147483647)

_QF = np.float32([0.1, 0.5, 0.9])
_QIDX = (_QF * (np.float32(N) - np.float32(1.0))).astype(np.float32)
RANK_LO = [int(x) for x in np.floor(_QIDX)]
RANK_HI = [int(x) for x in np.ceil(_QIDX)]
HIW = (_QIDX - np.floor(_QIDX)).astype(np.float32)
LOW = (np.float32(1.0) - HIW).astype(np.float32)

_mesh = plsc.VectorSubcoreMesh(core_axis_name="c", subcore_axis_name="s")


def _wid():
    return lax.axis_index("s") * NC + lax.axis_index("c")


def _keys(x):
    b = plsc.bitcast(x, jnp.int32)
    return jnp.where(b < 0, b ^ MASK31, b)


@functools.partial(
    pl.kernel,
    out_type=(
        jax.ShapeDtypeStruct((NW, L1_BINS * L), jnp.int32),
        jax.ShapeDtypeStruct((NW, 64), jnp.float32),
    ),
    mesh=_mesh,
    compiler_params=pltpu.CompilerParams(needs_layout_passes=False),
    scratch_types=[
        pltpu.VMEM((BLK,), jnp.float32),
        pltpu.VMEM((BLK,), jnp.float32),
        pltpu.VMEM((L1_BINS * L,), jnp.int32),
        pltpu.VMEM((64,), jnp.float32),
        pltpu.SemaphoreType.DMA,
        pltpu.SemaphoreType.DMA,
    ],
)
def _pass1(params, hist_out, mom_out, buf0, buf1, hist, momv, sem0, sem1):
    wid = _wid()
    base = wid * CHUNK
    lane = lax.iota(jnp.int32, L)
    lane_c = lane + jnp.int32(2048 * L)
    hi_mask = jnp.int32(-16)
    ones = jnp.ones((L,), jnp.int32)
    zeros16 = jnp.zeros((L,), jnp.int32)

    def zbody(i, _):
        for u in range(8):
            hist[pl.ds((i * 8 + u) * L, L)] = zeros16
        return 0
    lax.fori_loop(0, L1_BINS * L // (L * 8), zbody, 0)

    bufs = (buf0, buf1)
    sems = (sem0, sem1)
    cps = [None, None]
    cps[0] = pltpu.async_copy(params.at[pl.ds(base, BLK)], buf0, sem0)

    def make_body(buf):
        def body(i, carry):
            accs = [list(carry[:4]), list(carry[4:])]
            for u in range(2):
                x = buf[pl.ds((i + u) * L, L)]
                key = _keys(x)
                idx = ((key >> 16) & hi_mask) + lane_c
                plsc.addupdate_scatter(hist, [idx], ones)
                a = accs[u]
                a[0] = jnp.minimum(a[0], x)
                a[1] = jnp.maximum(a[1], x)
                a[2] = a[2] + x
                a[3] = a[3] + x * x
            return tuple(accs[0]) + tuple(accs[1])
        return body

    inf = jnp.full((L,), jnp.inf, jnp.float32)
    zf = jnp.zeros((L,), jnp.float32)
    carry = (inf, -inf, zf, zf, inf, -inf, zf, zf)
    for bkt in range(NBLK):
        cur = bkt % 2
        if bkt + 1 < NBLK:
            cps[1 - cur] = pltpu.async_copy(
                params.at[pl.ds(base + (bkt + 1) * BLK, BLK)],
                bufs[1 - cur], sems[1 - cur])
        cps[cur].wait()
        carry = plsc.parallel_loop(0, VPB, step=2, unroll=2,
                                   carry=carry)(make_body(bufs[cur]))

    momv[pl.ds(0, L)] = jnp.minimum(carry[0], carry[4])
    momv[pl.ds(L, L)] = jnp.maximum(carry[1], carry[5])
    momv[pl.ds(2 * L, L)] = carry[2] + carry[6]
    momv[pl.ds(3 * L, L)] = carry[3] + carry[7]
    pltpu.sync_copy(momv, mom_out.at[wid])
    pltpu.sync_copy(hist, hist_out.at[wid])


@functools.partial(
    pl.kernel,
    out_type=(
        jax.ShapeDtypeStruct((NW, 3 * L2_BINS * L), jnp.int32),
        jax.ShapeDtypeStruct((NW, 16 * L), jnp.int32),
    ),
    mesh=_mesh,
    compiler_params=pltpu.CompilerParams(needs_layout_passes=False),
    scratch_types=[
        pltpu.VMEM((BLK,), jnp.float32),
        pltpu.VMEM((BLK,), jnp.float32),
        pltpu.VMEM((3 * L2_BINS * L,), jnp.int32),
        pltpu.VMEM((16 * L,), jnp.int32),
        pltpu.VMEM((16,), jnp.int32),
        pltpu.VMEM((16,), jnp.float32),
        pltpu.SemaphoreType.DMA,
        pltpu.SemaphoreType.DMA,
    ],
)
def _pass2(params, scal_i, scal_f, hist_out, ehist_out,
           buf0, buf1, chist, ehist, sbi, sbf, sem0, sem1):
    wid = _wid()
    base = wid * CHUNK
    lane = lax.iota(jnp.int32, L)
    ones = jnp.ones((L,), jnp.int32)
    zeros16 = jnp.zeros((L,), jnp.int32)

    pltpu.sync_copy(scal_i, sbi)
    pltpu.sync_copy(scal_f, sbf)
    sv_i = sbi[pl.ds(0, L)]
    sv_f = sbf[pl.ds(0, L)]
    p12_0 = sv_i[0]
    p12_1 = sv_i[1]
    p12_2 = sv_i[2]
    mn = sv_f[0]
    inv_w = sv_f[1]

    def zbody(i, _):
        for u in range(8):
            chist[pl.ds((i * 8 + u) * L, L)] = zeros16
        return 0
    lax.fori_loop(0, 3 * L2_BINS * L // (L * 8), zbody, 0)
    def zebody(i, _):
        ehist[pl.ds(i * L, L)] = zeros16
        return 0
    lax.fori_loop(0, 16, zebody, 0)

    bufs = (buf0, buf1)
    sems = (sem0, sem1)
    cps = [None, None]
    cps[0] = pltpu.async_copy(params.at[pl.ds(base, BLK)], buf0, sem0)

    bin_mask = jnp.int32(1023 * L)

    def make_body(buf):
        def body(i, c):
            x = buf[pl.ds(i * L, L)]
            key = _keys(x)
            hi12 = key >> 20
            idx0 = ((key >> 6) & bin_mask) + lane
            plsc.addupdate_scatter(chist, [idx0], ones,
                                   mask=hi12 == p12_0)
            plsc.addupdate_scatter(chist, [idx0 + jnp.int32(L2_BINS * L)],
                                   ones, mask=hi12 == p12_1)
            plsc.addupdate_scatter(chist,
                                   [idx0 + jnp.int32(2 * L2_BINS * L)],
                                   ones, mask=hi12 == p12_2)
            t = (x - mn) * inv_w
            ie = jnp.clip(t.astype(jnp.int32), 0, 15)
            plsc.addupdate_scatter(ehist, [(ie << 4) + lane], ones)
            return c
        return body

    for bkt in range(NBLK):
        cur = bkt % 2
        if bkt + 1 < NBLK:
            cps[1 - cur] = pltpu.async_copy(
                params.at[pl.ds(base + (bkt + 1) * BLK, BLK)],
                bufs[1 - cur], sems[1 - cur])
        cps[cur].wait()
        plsc.parallel_loop(0, VPB, step=1, unroll=4,
                           carry=jnp.int32(0))(make_body(bufs[cur]))

    pltpu.sync_copy(ehist, ehist_out.at[wid])
    pltpu.sync_copy(chist, hist_out.at[wid])


@functools.partial(
    pl.kernel,
    out_type=(
        jax.ShapeDtypeStruct((NW, 3 * L2_BINS * L), jnp.int32),
        jax.ShapeDtypeStruct((NW, 48), jnp.int32),
    ),
    mesh=_mesh,
    compiler_params=pltpu.CompilerParams(needs_layout_passes=False),
    scratch_types=[
        pltpu.VMEM((BLK,), jnp.float32),
        pltpu.VMEM((BLK,), jnp.float32),
        pltpu.VMEM((3 * L2_BINS * L,), jnp.int32),
        pltpu.VMEM((48,), jnp.int32),
        pltpu.VMEM((16,), jnp.int32),
        pltpu.SemaphoreType.DMA,
        pltpu.SemaphoreType.DMA,
    ],
)
def _pass3(params, scal_i, hist_out, mink_out,
           buf0, buf1, chist, minkv, sbi, sem0, sem1):
    wid = _wid()
    base = wid * CHUNK
    lane = lax.iota(jnp.int32, L)
    ones = jnp.ones((L,), jnp.int32)
    zeros16 = jnp.zeros((L,), jnp.int32)

    pltpu.sync_copy(scal_i, sbi)
    sv_i = sbi[pl.ds(0, L)]
    p22_0 = sv_i[0]
    p22_1 = sv_i[1]
    p22_2 = sv_i[2]

    def zbody(i, _):
        for u in range(8):
            chist[pl.ds((i * 8 + u) * L, L)] = zeros16
        return 0
    lax.fori_loop(0, 3 * L2_BINS * L // (L * 8), zbody, 0)

    bufs = (buf0, buf1)
    sems = (sem0, sem1)
    cps = [None, None]
    cps[0] = pltpu.async_copy(params.at[pl.ds(base, BLK)], buf0, sem0)

    bin_mask = jnp.int32(1023)

    def make_body(buf):
        def body(i, carry):
            mks = [list(carry[:3]), list(carry[3:])]
            for u in range(2):
                x = buf[pl.ds((i + u) * L, L)]
                key = _keys(x)
                hi22 = key >> 10
                idx0 = ((key & bin_mask) << 4) + lane
                plsc.addupdate_scatter(chist, [idx0], ones,
                                       mask=hi22 == p22_0)
                plsc.addupdate_scatter(chist, [idx0 + jnp.int32(L2_BINS * L)],
                                       ones, mask=hi22 == p22_1)
                plsc.addupdate_scatter(chist,
                                       [idx0 + jnp.int32(2 * L2_BINS * L)],
                                       ones, mask=hi22 == p22_2)
                mk = mks[u]
                mk[0] = jnp.minimum(mk[0], jnp.where(hi22 > p22_0, key, I32MAX))
                mk[1] = jnp.minimum(mk[1], jnp.where(hi22 > p22_1, key, I32MAX))
                mk[2] = jnp.minimum(mk[2], jnp.where(hi22 > p22_2, key, I32MAX))
            return tuple(mks[0]) + tuple(mks[1])
        return body

    carry = (jnp.full((L,), I32MAX, jnp.int32),) * 6
    for bkt in range(NBLK):
        cur = bkt % 2
        if bkt + 1 < NBLK:
            cps[1 - cur] = pltpu.async_copy(
                params.at[pl.ds(base + (bkt + 1) * BLK, BLK)],
                bufs[1 - cur], sems[1 - cur])
        cps[cur].wait()
        carry = plsc.parallel_loop(0, VPB, step=2, unroll=2,
                                   carry=carry)(make_body(bufs[cur]))

    minkv[pl.ds(0, L)] = jnp.minimum(carry[0], carry[3])
    minkv[pl.ds(L, L)] = jnp.minimum(carry[1], carry[4])
    minkv[pl.ds(2 * L, L)] = jnp.minimum(carry[2], carry[5])
    pltpu.sync_copy(minkv, mink_out.at[wid])
    pltpu.sync_copy(chist, hist_out.at[wid])


def _heads_body(ecnt_ref, scal_ref,
                dW1_ref, db1_ref, dg_ref, dbeta_ref, dW2_ref, db2_ref,
                fW1_ref, fb1_ref, fg_ref, fbeta_ref, fW2_ref, fb2_ref,
                dens_ref, fish_ref, ent_ref):
    counts = ecnt_ref[0:1, :]
    probs = counts * jnp.float32(1.0 / N)
    logp = jnp.log(jnp.where(probs > 0, probs, jnp.float32(1.0)))
    ent = -jnp.sum(jnp.where(probs > 0, probs * logp, jnp.float32(0.0)))
    norm_ent = ent * jnp.float32(1.4426950408889634 / 4.0)
    ent_ref[...] = jnp.reshape(norm_ent, (1, 1))

    lanes = lax.broadcasted_iota(jnp.int32, (8, 128), 1)
    feat_f = jnp.broadcast_to(scal_ref[0:1, :], (8, 128))
    feat_d = jnp.where(lanes == 7, norm_ent, feat_f)

    def head(feat, W1, b1, g, beta, W2, b2):
        h = lax.dot_general(feat, W1[...], (((1,), (1,)), ((), ())),
                            preferred_element_type=jnp.float32)
        h = h + b1[0:1, :]
        mu = jnp.mean(h, axis=-1, keepdims=True)
        var = jnp.mean((h - mu) ** 2, axis=-1, keepdims=True)
        h = (h - mu) / jnp.sqrt(var + jnp.float32(1e-5)) * g[0:1, :] + beta[0:1, :]
        h = jnp.float32(0.5) * h * (jnp.float32(1.0) +
                                    lax.erf(h * jnp.float32(0.7071067811865476)))
        o = lax.dot_general(h, W2[...], (((1,), (1,)), ((), ())),
                            preferred_element_type=jnp.float32)
        return o[0:1, 0:1] + b2[0:1, 0:1]

    of = head(feat_f, fW1_ref, fb1_ref, fg_ref, fbeta_ref, fW2_ref, fb2_ref)
    fish_ref[...] = (jnp.maximum(of, 0.0) +
                     jnp.log1p(jnp.exp(-jnp.abs(of))))

    od = head(feat_d, dW1_ref, db1_ref, dg_ref, dbeta_ref, dW2_ref, db2_ref)
    dens_ref[...] = jnp.float32(1.0) / (jnp.float32(1.0) + jnp.exp(-od))


_heads = pl.pallas_call(
    _heads_body,
    out_shape=(
        jax.ShapeDtypeStruct((1, 1), jnp.float32),
        jax.ShapeDtypeStruct((1, 1), jnp.float32),
        jax.ShapeDtypeStruct((1, 1), jnp.float32),
    ),
)


def _first_ge(cum, target):
    return jnp.sum((cum < target).astype(jnp.int32), axis=-1)


def _unkey(k):
    return lax.bitcast_convert_type(jnp.where(k < 0, k ^ MASK31, k),
                                    jnp.float32)


def kernel(parameters, dW1, db1, dg, dbeta, dW2, db2,
           fW1, fb1, fg, fbeta, fW2, fb2):
    ranks = jnp.array(RANK_LO, jnp.int32)

    hist1, mom = _pass1(parameters)
    mom = mom.reshape(NW, 4, L)
    mn = jnp.min(mom[:, 0, :])
    mx = jnp.max(mom[:, 1, :])
    sm = jnp.sum(mom[:, 2, :])
    sq = jnp.sum(mom[:, 3, :])
    mean = sm / jnp.float32(N)
    var = (sq - sm * sm / jnp.float32(N)) / jnp.float32(N - 1)
    std = jnp.sqrt(var)

    h1 = jnp.sum(hist1.reshape(NW, L1_BINS, L), axis=(0, 2))
    cum1 = jnp.cumsum(h1)
    b1 = _first_ge(cum1[None, :], (ranks + 1)[:, None])
    cb1 = cum1[b1] - h1[b1]
    r1 = ranks - cb1
    p12 = b1 - 2048

    scal_i2 = jnp.zeros((16,), jnp.int32).at[0:3].set(p12)
    inv_w = jnp.float32(16.0) / (mx - mn)
    scal_f2 = jnp.zeros((16,), jnp.float32).at[0].set(mn).at[1].set(inv_w)

    hist2, ehist = _pass2(parameters, scal_i2, scal_f2)
    h2 = jnp.sum(hist2.reshape(NW, 3, L2_BINS, L), axis=(0, 3))
    cum2 = jnp.cumsum(h2, axis=1)
    b2 = _first_ge(cum2, (r1 + 1)[:, None])
    h2b = jnp.take_along_axis(h2, b2[:, None], axis=1)[:, 0]
    c2b = jnp.take_along_axis(cum2, b2[:, None], axis=1)[:, 0] - h2b
    r2 = r1 - c2b
    p22 = p12 * 1024 + b2

    scal_i3 = jnp.zeros((16,), jnp.int32).at[0:3].set(p22)
    hist3, mink = _pass3(parameters, scal_i3)
    h3 = jnp.sum(hist3.reshape(NW, 3, L2_BINS, L), axis=(0, 3))
    cum3 = jnp.cumsum(h3, axis=1)
    b3 = _first_ge(cum3, (r2 + 1)[:, None])
    c3 = jnp.take_along_axis(cum3, b3[:, None], axis=1)[:, 0]
    K = p22 * 1024 + b3
    v_lo = _unkey(K)

    cnt_le = cb1 + c2b + c3
    need_next = cnt_le < ranks + 2
    iota3 = lax.broadcasted_iota(jnp.int32, (3, L2_BINS), 1)
    above = (iota3 > b3[:, None]) & (h3 > 0)
    bnext = jnp.min(jnp.where(above, iota3, jnp.int32(L2_BINS)), axis=1)
    mk = jnp.min(mink.reshape(NW, 3, L), axis=(0, 2))
    Kn = jnp.where(bnext < L2_BINS, p22 * 1024 + bnext, mk)
    v_hi = jnp.where(need_next, _unkey(Kn), v_lo)

    qv = v_lo * jnp.asarray(LOW) + v_hi * jnp.asarray(HIW)

    scal = jnp.zeros((1, 128), jnp.float32)
    scal = scal.at[0, 0].set(mean).at[0, 1].set(std).at[0, 2].set(mx)
    scal = scal.at[0, 3].set(mn).at[0, 4].set(qv[0]).at[0, 5].set(qv[1])
    scal = scal.at[0, 6].set(qv[2])

    ecnt = jnp.sum(ehist.reshape(NW, 16, L), axis=(0, 2)).astype(jnp.float32)[None, :]

    dens, fish, nent = _heads(
        ecnt, scal,
        dW1, db1[None, :], dg[None, :], dbeta[None, :], dW2, db2[None, :],
        fW1, fb1[None, :], fg[None, :], fbeta[None, :], fW2, fb2[None, :])
    return (dens.reshape(()), fish.reshape(()), nent.reshape(()))
